# baseline (speedup 1.0000x reference)
import functools

import jax
import jax.numpy as jnp
from jax import lax
from jax.experimental import pallas as pl
from jax.experimental.pallas import tpu as pltpu
from jax.experimental.pallas import tpu_sc as plsc

N = 10000
E = 320000
F = 512
H = 128
NLAYERS = 3

NC = 2
NS = 16
NW = NC * NS
EPW = E // NW
C = 80
NCHUNK = EPW // C
UNROLL = 6
NB = 3
ZR = 624
ZREM = N - NS * ZR
HV = H // 16
TPAD = 104


def _sc_agg_body(x_hbm, edata_hbm, emb_hbm, zero_hbm, out_hbm,
                 i0_v, i1_v, i2_v, i3_v, i4_v, i5_v, b0_v, b1_v, b2_v,
                 emb_sh, agg_sh, sem_i, sem_e, sem_x, sem_s):
    c = lax.axis_index("c")
    s = lax.axis_index("s")
    w = c * NS + s
    idxs = (i0_v, i1_v, i2_v, i3_v, i4_v, i5_v)
    bufs = (b0_v, b1_v, b2_v)

    def start_idx(k, j):
        pltpu.async_copy(edata_hbm.at[w, k], idxs[j % UNROLL], sem_i)

    def start_e(j):
        pltpu.async_copy(emb_hbm.at[idxs[j % UNROLL].at[0]], bufs[j % NB],
                         sem_e)

    def start_x(j):
        pltpu.async_copy(x_hbm.at[idxs[j % UNROLL].at[1]], bufs[j % NB],
                         sem_x, add=True)

    def start_scat(j):
        pltpu.async_copy(bufs[j % NB], agg_sh.at[idxs[j % UNROLL].at[2]],
                         sem_s, add=True)

    def drain(sem):
        pltpu.make_async_copy(x_hbm.at[pl.ds(0, C)], b0_v, sem).wait()

    def drain_idx():
        pltpu.make_async_copy(edata_hbm.at[0, 0], i0_v, sem_i).wait()

    def relu_buf(buf):
        def relu_row(r, carry):
            for j in range(HV):
                v = buf[r, pl.ds(j * 16, 16)]
                buf[r, pl.ds(j * 16, 16)] = jnp.maximum(v, 0.0)
            return carry
        lax.fori_loop(0, C, relu_row, 0)

    def slot(k, j):
        drain(sem_x)

        @pl.when(k + 1 < NCHUNK)
        def _next_x():
            drain(sem_e)
            start_x(j + 1)

        relu_buf(bufs[j % NB])

        @pl.when(k > 0)
        def _prev_scat():
            drain(sem_s)

        start_scat(j)

        @pl.when(k + 2 < NCHUNK)
        def _next_e():
            drain_idx()
            start_e(j + 2)

        @pl.when(k + 3 < NCHUNK)
        def _next_idx():
            start_idx(k + 3, j + 3)

    pltpu.sync_copy(edata_hbm.at[w, 0], i0_v)
    start_idx(1, 1)
    start_idx(2, 2)

    @pl.when(s == 0)
    def _load_emb():
        pltpu.sync_copy(emb_hbm, emb_sh)

    zbase = pl.multiple_of(s * ZR, 8)
    pltpu.sync_copy(zero_hbm.at[pl.ds(zbase, ZR)],
                    agg_sh.at[pl.ds(zbase, ZR)])

    @pl.when(s == NS - 1)
    def _zero_rem():
        pltpu.sync_copy(zero_hbm.at[pl.ds(NS * ZR, ZREM)],
                        agg_sh.at[pl.ds(NS * ZR, ZREM)])

    plsc.subcore_barrier()

    start_e(0)
    drain(sem_e)
    start_x(0)
    drain_idx()
    start_e(1)

    nmain = NCHUNK // UNROLL

    def body(m, carry):
        k0 = m * UNROLL
        for j in range(UNROLL):
            slot(k0 + j, j)
        return carry

    lax.fori_loop(0, nmain, body, 0)
    for j in range(NCHUNK - nmain * UNROLL):
        slot(nmain * UNROLL + j, j)

    drain(sem_s)
    plsc.subcore_barrier()

    wbase = pl.multiple_of(s * ZR, 8)
    pltpu.sync_copy(agg_sh.at[pl.ds(wbase, ZR)],
                    out_hbm.at[c, pl.ds(wbase, ZR)])

    @pl.when(s == NS - 1)
    def _wb_rem():
        pltpu.sync_copy(agg_sh.at[pl.ds(NS * ZR, ZREM)],
                        out_hbm.at[c, pl.ds(NS * ZR, ZREM)])


_sc_agg = pl.kernel(
    _sc_agg_body,
    out_type=jax.ShapeDtypeStruct((NC, N, H), jnp.float32),
    mesh=plsc.VectorSubcoreMesh(core_axis_name="c", subcore_axis_name="s"),
    scratch_types=[
        pltpu.VMEM((3, C), jnp.int32),
        pltpu.VMEM((3, C), jnp.int32),
        pltpu.VMEM((3, C), jnp.int32),
        pltpu.VMEM((3, C), jnp.int32),
        pltpu.VMEM((3, C), jnp.int32),
        pltpu.VMEM((3, C), jnp.int32),
        pltpu.VMEM((C, H), jnp.float32),
        pltpu.VMEM((C, H), jnp.float32),
        pltpu.VMEM((C, H), jnp.float32),
        pltpu.VMEM_SHARED((TPAD, H), jnp.float32),
        pltpu.VMEM_SHARED((N, H), jnp.float32),
        pltpu.SemaphoreType.DMA,
        pltpu.SemaphoreType.DMA,
        pltpu.SemaphoreType.DMA,
        pltpu.SemaphoreType.DMA,
    ],
)


BR = 1000


def _proj_body(p_ref, wp_ref, bp_ref, o_ref):
    o_ref[...] = jnp.dot(p_ref[...], wp_ref[...],
                         preferred_element_type=jnp.float32) + bp_ref[...]


_proj = pl.pallas_call(
    _proj_body,
    grid=(N // BR,),
    in_specs=[
        pl.BlockSpec((BR, F), lambda i: (i, 0)),
        pl.BlockSpec((F, H), lambda i: (0, 0)),
        pl.BlockSpec((1, H), lambda i: (0, 0)),
    ],
    out_specs=pl.BlockSpec((BR, H), lambda i: (i, 0)),
    out_shape=jax.ShapeDtypeStruct((N, H), jnp.float32),
)


def _mlp_body(x_ref, agg_ref, w1_ref, b1_ref, w2_ref, b2_ref, g_ref, be_ref,
              o_ref, *, last):
    x = x_ref[...]
    h0 = x + agg_ref[0] + agg_ref[1]
    t = jnp.maximum(jnp.dot(h0, w1_ref[...],
                            preferred_element_type=jnp.float32) + b1_ref[...],
                    0.0)
    h = jnp.dot(t, w2_ref[...],
                preferred_element_type=jnp.float32) + b2_ref[...] + x
    if last:
        mu = jnp.mean(h, axis=-1, keepdims=True)
        var = jnp.mean((h - mu) ** 2, axis=-1, keepdims=True)
        h = (h - mu) * lax.rsqrt(var + 1e-5) * g_ref[...] + be_ref[...]
    o_ref[...] = h


def _make_mlp(last):
    return pl.pallas_call(
        functools.partial(_mlp_body, last=last),
        grid=(N // BR,),
        in_specs=[
            pl.BlockSpec((BR, H), lambda i: (i, 0)),
            pl.BlockSpec((NC, BR, H), lambda i: (0, i, 0)),
            pl.BlockSpec((H, H), lambda i: (0, 0)),
            pl.BlockSpec((1, H), lambda i: (0, 0)),
            pl.BlockSpec((H, H), lambda i: (0, 0)),
            pl.BlockSpec((1, H), lambda i: (0, 0)),
            pl.BlockSpec((1, H), lambda i: (0, 0)),
            pl.BlockSpec((1, H), lambda i: (0, 0)),
        ],
        out_specs=pl.BlockSpec((BR, H), lambda i: (i, 0)),
        out_shape=jax.ShapeDtypeStruct((N, H), jnp.float32),
    )


_mlp_mid = _make_mlp(False)
_mlp_last = _make_mlp(True)


def kernel(peptide_feature, edge_index, edge_attr, Wp, bp, W1, b1, W2, b2,
           emb_table, gamma, beta):
    src = edge_index[0]
    dst = edge_index[1]
    tt = edge_attr[:, 0]
    edata = jnp.stack([tt, src, dst]).reshape(3, NW, NCHUNK, C)
    edata = edata.transpose(1, 2, 0, 3)
    emb_p = jnp.zeros((TPAD, H), jnp.float32).at[:100].set(emb_table)
    zeros = jnp.zeros((N, H), jnp.float32)
    bp2 = bp.reshape(1, H)
    g2 = gamma.reshape(1, H)
    be2 = beta.reshape(1, H)

    x = _proj(peptide_feature, Wp, bp2)
    for i in range(NLAYERS):
        agg = _sc_agg(x, edata, emb_p, zeros)
        mlp = _mlp_last if i == NLAYERS - 1 else _mlp_mid
        x = mlp(x, agg, W1[i], b1[i].reshape(1, H), W2[i],
                b2[i].reshape(1, H), g2, be2)
    return x


# speedup vs baseline: 1.2844x; 1.2844x over previous
import functools

import jax
import jax.numpy as jnp
from jax import lax
from jax.experimental import pallas as pl
from jax.experimental.pallas import tpu as pltpu
from jax.experimental.pallas import tpu_sc as plsc

N = 10000
E = 320000
F = 512
H = 128
NLAYERS = 3

NC = 2
NS = 16
NW = NC * NS
EPW = E // NW
C = 80
NCHUNK = EPW // C
UNROLL = 6
NB = 3
ZR = 624
ZREM = N - NS * ZR
HV = H // 16
TPAD = 104


def _sc_agg_body(x_hbm, edata_hbm, emb_hbm, zero_hbm, out_hbm,
                 i0_v, i1_v, i2_v, i3_v, i4_v, i5_v, b0_v, b1_v, b2_v,
                 emb_sh, agg_sh, sem_i, sem_e, sem_x, sem_x2, sem_s):
    c = lax.axis_index("c")
    s = lax.axis_index("s")
    w = c * NS + s
    idxs = (i0_v, i1_v, i2_v, i3_v, i4_v, i5_v)
    bufs = (b0_v, b1_v, b2_v)

    def start_idx(k, j):
        pltpu.async_copy(edata_hbm.at[w, k], idxs[j % UNROLL], sem_i)

    def start_e(j):
        pltpu.async_copy(emb_sh.at[idxs[j % UNROLL].at[0]], bufs[j % NB],
                         sem_e)

    def start_x(j):
        sem = sem_x if j % 2 == 0 else sem_x2
        pltpu.async_copy(x_hbm.at[idxs[j % UNROLL].at[1]], bufs[j % NB],
                         sem, add=True)

    def start_scat(j):
        pltpu.async_copy(bufs[j % NB], agg_sh.at[idxs[j % UNROLL].at[2]],
                         sem_s, add=True)

    def drain(sem):
        pltpu.make_async_copy(x_hbm.at[pl.ds(0, C)], b0_v, sem).wait()

    def drain_idx():
        pltpu.make_async_copy(edata_hbm.at[0, 0], i0_v, sem_i).wait()

    def relu_buf(buf):
        def relu_row(r, carry):
            for j in range(HV):
                v = buf[r, pl.ds(j * 16, 16)]
                buf[r, pl.ds(j * 16, 16)] = jnp.maximum(v, 0.0)
            return carry
        lax.fori_loop(0, C, relu_row, 0)

    def slot(k, j):
        @pl.when(k + 2 < NCHUNK)
        def _idx_arrived():
            drain_idx()

        drain(sem_x if j % 2 == 0 else sem_x2)

        @pl.when(k > 0)
        def _prev_scat():
            drain(sem_s)

        @pl.when(k + 2 < NCHUNK)
        def _next_e():
            start_e(j + 2)

        relu_buf(bufs[j % NB])

        start_scat(j)

        @pl.when(k + 2 < NCHUNK)
        def _next_x():
            drain(sem_e)
            start_x(j + 2)

        @pl.when(k + 3 < NCHUNK)
        def _next_idx():
            start_idx(k + 3, j + 3)

    pltpu.sync_copy(edata_hbm.at[w, 0], i0_v)
    start_idx(1, 1)
    start_idx(2, 2)

    @pl.when(s == 0)
    def _load_emb():
        pltpu.sync_copy(emb_hbm, emb_sh)

    zbase = pl.multiple_of(s * ZR, 8)
    pltpu.sync_copy(zero_hbm.at[pl.ds(zbase, ZR)],
                    agg_sh.at[pl.ds(zbase, ZR)])

    @pl.when(s == NS - 1)
    def _zero_rem():
        pltpu.sync_copy(zero_hbm.at[pl.ds(NS * ZR, ZREM)],
                        agg_sh.at[pl.ds(NS * ZR, ZREM)])

    plsc.subcore_barrier()

    start_e(0)
    drain(sem_e)
    start_x(0)
    drain_idx()
    start_e(1)
    drain(sem_e)
    start_x(1)

    nmain = NCHUNK // UNROLL

    def body(m, carry):
        k0 = m * UNROLL
        for j in range(UNROLL):
            slot(k0 + j, j)
        return carry

    lax.fori_loop(0, nmain, body, 0)
    for j in range(NCHUNK - nmain * UNROLL):
        slot(nmain * UNROLL + j, j)

    drain(sem_s)
    plsc.subcore_barrier()

    wbase = pl.multiple_of(s * ZR, 8)
    pltpu.sync_copy(agg_sh.at[pl.ds(wbase, ZR)],
                    out_hbm.at[c, pl.ds(wbase, ZR)])

    @pl.when(s == NS - 1)
    def _wb_rem():
        pltpu.sync_copy(agg_sh.at[pl.ds(NS * ZR, ZREM)],
                        out_hbm.at[c, pl.ds(NS * ZR, ZREM)])


_sc_agg = pl.kernel(
    _sc_agg_body,
    out_type=jax.ShapeDtypeStruct((NC, N, H), jnp.float32),
    mesh=plsc.VectorSubcoreMesh(core_axis_name="c", subcore_axis_name="s"),
    scratch_types=[
        pltpu.VMEM((3, C), jnp.int32),
        pltpu.VMEM((3, C), jnp.int32),
        pltpu.VMEM((3, C), jnp.int32),
        pltpu.VMEM((3, C), jnp.int32),
        pltpu.VMEM((3, C), jnp.int32),
        pltpu.VMEM((3, C), jnp.int32),
        pltpu.VMEM((C, H), jnp.float32),
        pltpu.VMEM((C, H), jnp.float32),
        pltpu.VMEM((C, H), jnp.float32),
        pltpu.VMEM_SHARED((TPAD, H), jnp.float32),
        pltpu.VMEM_SHARED((N, H), jnp.float32),
        pltpu.SemaphoreType.DMA,
        pltpu.SemaphoreType.DMA,
        pltpu.SemaphoreType.DMA,
        pltpu.SemaphoreType.DMA,
        pltpu.SemaphoreType.DMA,
    ],
)


BR = 1000


def _proj_body(p_ref, wp_ref, bp_ref, o_ref):
    o_ref[...] = jnp.dot(p_ref[...], wp_ref[...],
                         preferred_element_type=jnp.float32) + bp_ref[...]


_proj = pl.pallas_call(
    _proj_body,
    grid=(N // BR,),
    in_specs=[
        pl.BlockSpec((BR, F), lambda i: (i, 0)),
        pl.BlockSpec((F, H), lambda i: (0, 0)),
        pl.BlockSpec((1, H), lambda i: (0, 0)),
    ],
    out_specs=pl.BlockSpec((BR, H), lambda i: (i, 0)),
    out_shape=jax.ShapeDtypeStruct((N, H), jnp.float32),
)


def _mlp_body(x_ref, agg_ref, w1_ref, b1_ref, w2_ref, b2_ref, g_ref, be_ref,
              o_ref, *, last):
    x = x_ref[...]
    h0 = x + agg_ref[0] + agg_ref[1]
    t = jnp.maximum(jnp.dot(h0, w1_ref[...],
                            preferred_element_type=jnp.float32) + b1_ref[...],
                    0.0)
    h = jnp.dot(t, w2_ref[...],
                preferred_element_type=jnp.float32) + b2_ref[...] + x
    if last:
        mu = jnp.mean(h, axis=-1, keepdims=True)
        var = jnp.mean((h - mu) ** 2, axis=-1, keepdims=True)
        h = (h - mu) * lax.rsqrt(var + 1e-5) * g_ref[...] + be_ref[...]
    o_ref[...] = h


def _make_mlp(last):
    return pl.pallas_call(
        functools.partial(_mlp_body, last=last),
        grid=(N // BR,),
        in_specs=[
            pl.BlockSpec((BR, H), lambda i: (i, 0)),
            pl.BlockSpec((NC, BR, H), lambda i: (0, i, 0)),
            pl.BlockSpec((H, H), lambda i: (0, 0)),
            pl.BlockSpec((1, H), lambda i: (0, 0)),
            pl.BlockSpec((H, H), lambda i: (0, 0)),
            pl.BlockSpec((1, H), lambda i: (0, 0)),
            pl.BlockSpec((1, H), lambda i: (0, 0)),
            pl.BlockSpec((1, H), lambda i: (0, 0)),
        ],
        out_specs=pl.BlockSpec((BR, H), lambda i: (i, 0)),
        out_shape=jax.ShapeDtypeStruct((N, H), jnp.float32),
    )


_mlp_mid = _make_mlp(False)
_mlp_last = _make_mlp(True)


def kernel(peptide_feature, edge_index, edge_attr, Wp, bp, W1, b1, W2, b2,
           emb_table, gamma, beta):
    src = edge_index[0]
    dst = edge_index[1]
    tt = edge_attr[:, 0]
    edata = jnp.stack([tt, src, dst]).reshape(3, NW, NCHUNK, C)
    edata = edata.transpose(1, 2, 0, 3)
    emb_p = jnp.zeros((TPAD, H), jnp.float32).at[:100].set(emb_table)
    zeros = jnp.zeros((N, H), jnp.float32)
    bp2 = bp.reshape(1, H)
    g2 = gamma.reshape(1, H)
    be2 = beta.reshape(1, H)

    x = _proj(peptide_feature, Wp, bp2)
    for i in range(NLAYERS):
        agg = _sc_agg(x, edata, emb_p, zeros)
        mlp = _mlp_last if i == NLAYERS - 1 else _mlp_mid
        x = mlp(x, agg, W1[i], b1[i].reshape(1, H), W2[i],
                b2[i].reshape(1, H), g2, be2)
    return x


# speedup vs baseline: 1.5688x; 1.2214x over previous
import functools

import jax
import jax.numpy as jnp
from jax import lax
from jax.experimental import pallas as pl
from jax.experimental.pallas import tpu as pltpu
from jax.experimental.pallas import tpu_sc as plsc

N = 10000
E = 320000
F = 512
H = 128
NLAYERS = 3

NC = 2
NS = 16
NW = NC * NS
EPW = E // NW
C = 80
NCHUNK = EPW // C
UNROLL = 6
NB = 3
ZR = 624
ZREM = N - NS * ZR
HV = H // 16
TPAD = 104


def _sc_agg_body(x_hbm, edata_hbm, emb_hbm, zero_hbm, out_hbm,
                 i0_v, i1_v, i2_v, i3_v, i4_v, i5_v, b0_v, b1_v, b2_v,
                 emb_sh, agg_sh, sem_i, sem_e, sem_x, sem_s):
    c = lax.axis_index("c")
    s = lax.axis_index("s")
    w = c * NS + s
    idxs = (i0_v, i1_v, i2_v, i3_v, i4_v, i5_v)
    bufs = (b0_v, b1_v, b2_v)

    def start_idx(k, j):
        pltpu.async_copy(edata_hbm.at[w, k], idxs[j % UNROLL], sem_i)

    def start_e(j):
        pltpu.async_copy(emb_sh.at[idxs[j % UNROLL].at[0]], bufs[j % NB],
                         sem_e)

    def start_x(j):
        pltpu.async_copy(x_hbm.at[idxs[j % UNROLL].at[1]], bufs[j % NB],
                         sem_x, add=True)

    def start_scat(j):
        pltpu.async_copy(bufs[j % NB], agg_sh.at[idxs[j % UNROLL].at[2]],
                         sem_s, add=True)

    def drain(sem):
        pltpu.make_async_copy(x_hbm.at[pl.ds(0, C)], b0_v, sem).wait()

    def drain_idx():
        pltpu.make_async_copy(edata_hbm.at[0, 0], i0_v, sem_i).wait()

    def relu_buf(buf, lo, hi):
        def relu_row(r, carry):
            for j in range(HV):
                v = buf[r, pl.ds(j * 16, 16)]
                buf[r, pl.ds(j * 16, 16)] = jnp.maximum(v, 0.0)
            return carry
        lax.fori_loop(lo, hi, relu_row, 0)

    def slot(k, j):
        drain(sem_x)
        relu_buf(bufs[j % NB], 0, C // 2)

        @pl.when(k + 1 < NCHUNK)
        def _next_x():
            drain(sem_e)
            start_x(j + 1)

        relu_buf(bufs[j % NB], C // 2, C)

        @pl.when(k > 0)
        def _prev_scat():
            drain(sem_s)

        start_scat(j)

        @pl.when(k + 2 < NCHUNK)
        def _next_e():
            drain_idx()
            start_e(j + 2)

        @pl.when(k + 3 < NCHUNK)
        def _next_idx():
            start_idx(k + 3, j + 3)

    pltpu.sync_copy(edata_hbm.at[w, 0], i0_v)
    start_idx(1, 1)
    start_idx(2, 2)

    @pl.when(s == 0)
    def _load_emb():
        pltpu.sync_copy(emb_hbm, emb_sh)

    zbase = pl.multiple_of(s * ZR, 8)
    pltpu.sync_copy(zero_hbm.at[pl.ds(zbase, ZR)],
                    agg_sh.at[pl.ds(zbase, ZR)])

    @pl.when(s == NS - 1)
    def _zero_rem():
        pltpu.sync_copy(zero_hbm.at[pl.ds(NS * ZR, ZREM)],
                        agg_sh.at[pl.ds(NS * ZR, ZREM)])

    plsc.subcore_barrier()

    start_e(0)
    drain(sem_e)
    start_x(0)
    drain_idx()
    start_e(1)

    nmain = NCHUNK // UNROLL

    def body(m, carry):
        k0 = m * UNROLL
        for j in range(UNROLL):
            slot(k0 + j, j)
        return carry

    lax.fori_loop(0, nmain, body, 0)
    for j in range(NCHUNK - nmain * UNROLL):
        slot(nmain * UNROLL + j, j)

    drain(sem_s)
    plsc.subcore_barrier()

    wbase = pl.multiple_of(s * ZR, 8)
    pltpu.sync_copy(agg_sh.at[pl.ds(wbase, ZR)],
                    out_hbm.at[c, pl.ds(wbase, ZR)])

    @pl.when(s == NS - 1)
    def _wb_rem():
        pltpu.sync_copy(agg_sh.at[pl.ds(NS * ZR, ZREM)],
                        out_hbm.at[c, pl.ds(NS * ZR, ZREM)])


_sc_agg = pl.kernel(
    _sc_agg_body,
    out_type=jax.ShapeDtypeStruct((NC, N, H), jnp.float32),
    mesh=plsc.VectorSubcoreMesh(core_axis_name="c", subcore_axis_name="s"),
    scratch_types=[
        pltpu.VMEM((3, C), jnp.int32),
        pltpu.VMEM((3, C), jnp.int32),
        pltpu.VMEM((3, C), jnp.int32),
        pltpu.VMEM((3, C), jnp.int32),
        pltpu.VMEM((3, C), jnp.int32),
        pltpu.VMEM((3, C), jnp.int32),
        pltpu.VMEM((C, H), jnp.float32),
        pltpu.VMEM((C, H), jnp.float32),
        pltpu.VMEM((C, H), jnp.float32),
        pltpu.VMEM_SHARED((TPAD, H), jnp.float32),
        pltpu.VMEM_SHARED((N, H), jnp.float32),
        pltpu.SemaphoreType.DMA,
        pltpu.SemaphoreType.DMA,
        pltpu.SemaphoreType.DMA,
        pltpu.SemaphoreType.DMA,
    ],
)


BR = 1000


def _proj_body(p_ref, wp_ref, bp_ref, o_ref):
    o_ref[...] = jnp.dot(p_ref[...], wp_ref[...],
                         preferred_element_type=jnp.float32) + bp_ref[...]


_proj = pl.pallas_call(
    _proj_body,
    grid=(N // BR,),
    in_specs=[
        pl.BlockSpec((BR, F), lambda i: (i, 0)),
        pl.BlockSpec((F, H), lambda i: (0, 0)),
        pl.BlockSpec((1, H), lambda i: (0, 0)),
    ],
    out_specs=pl.BlockSpec((BR, H), lambda i: (i, 0)),
    out_shape=jax.ShapeDtypeStruct((N, H), jnp.float32),
)


def _mlp_body(x_ref, agg_ref, w1_ref, b1_ref, w2_ref, b2_ref, g_ref, be_ref,
              o_ref, *, last):
    x = x_ref[...]
    h0 = x + agg_ref[0] + agg_ref[1]
    t = jnp.maximum(jnp.dot(h0, w1_ref[...],
                            preferred_element_type=jnp.float32) + b1_ref[...],
                    0.0)
    h = jnp.dot(t, w2_ref[...],
                preferred_element_type=jnp.float32) + b2_ref[...] + x
    if last:
        mu = jnp.mean(h, axis=-1, keepdims=True)
        var = jnp.mean((h - mu) ** 2, axis=-1, keepdims=True)
        h = (h - mu) * lax.rsqrt(var + 1e-5) * g_ref[...] + be_ref[...]
    o_ref[...] = h


def _make_mlp(last):
    return pl.pallas_call(
        functools.partial(_mlp_body, last=last),
        grid=(N // BR,),
        in_specs=[
            pl.BlockSpec((BR, H), lambda i: (i, 0)),
            pl.BlockSpec((NC, BR, H), lambda i: (0, i, 0)),
            pl.BlockSpec((H, H), lambda i: (0, 0)),
            pl.BlockSpec((1, H), lambda i: (0, 0)),
            pl.BlockSpec((H, H), lambda i: (0, 0)),
            pl.BlockSpec((1, H), lambda i: (0, 0)),
            pl.BlockSpec((1, H), lambda i: (0, 0)),
            pl.BlockSpec((1, H), lambda i: (0, 0)),
        ],
        out_specs=pl.BlockSpec((BR, H), lambda i: (i, 0)),
        out_shape=jax.ShapeDtypeStruct((N, H), jnp.float32),
    )


_mlp_mid = _make_mlp(False)
_mlp_last = _make_mlp(True)


def kernel(peptide_feature, edge_index, edge_attr, Wp, bp, W1, b1, W2, b2,
           emb_table, gamma, beta):
    src = edge_index[0]
    dst = edge_index[1]
    tt = edge_attr[:, 0]
    edata = jnp.stack([tt, src, dst]).reshape(3, NW, NCHUNK, C)
    edata = edata.transpose(1, 2, 0, 3)
    emb_p = jnp.zeros((TPAD, H), jnp.float32).at[:100].set(emb_table)
    zeros = jnp.zeros((N, H), jnp.float32)
    bp2 = bp.reshape(1, H)
    g2 = gamma.reshape(1, H)
    be2 = beta.reshape(1, H)

    x = _proj(peptide_feature, Wp, bp2)
    for i in range(NLAYERS):
        agg = _sc_agg(x, edata, emb_p, zeros)
        mlp = _mlp_last if i == NLAYERS - 1 else _mlp_mid
        x = mlp(x, agg, W1[i], b1[i].reshape(1, H), W2[i],
                b2[i].reshape(1, H), g2, be2)
    return x
